# baseline (speedup 1.0000x reference)
import functools

import jax
import jax.numpy as jnp
from jax import lax
from jax.experimental import pallas as pl
from jax.experimental.pallas import tpu as pltpu
from jax.experimental.pallas import tpu_sc as plsc

_L = 16
_RING = 4
_W = 128
_SUB = 128


def _run_table(tab_hbm, tail0, sv_v, sv2_v, pl_v, starts_v, myrows_v,
               tail_v, bufs, sems, sem_s, stage_hbm, *, bpw):
    iota = lax.iota(jnp.int32, _L)
    nch = bpw // _L

    sv2_v[pl.ds(0, _L)] = jnp.full((_L,), -1, jnp.int32)

    def shift(k, c):
        sv2_v[pl.ds(k * _L + 1, _L)] = sv_v[pl.ds(k * _L, _L)]
        return c
    lax.fori_loop(0, nch, shift, 0)

    def fill(k, c):
        starts_v[pl.ds(k * _L, _L)] = jnp.full((_L,), bpw, jnp.int32)
        return c
    lax.fori_loop(0, (bpw + 2 * _L) // _L, fill, 0)

    def detect(k, cnt):
        w = sv_v[pl.ds(k * _L, _L)] >> 7
        wp = sv2_v[pl.ds(k * _L, _L)] >> 7
        neq = w != wp
        plsc.store_compressed(starts_v.at[pl.ds(cnt, _L)],
                              k * _L + iota, mask=neq)
        npc = plsc.all_reduce_population_count(neq)
        return cnt + npc[0]
    kruns = lax.fori_loop(0, nch, detect, 0)
    kp = ((kruns + _RING - 1) // _RING) * _RING

    def win_of(t):
        r = sv_v[pl.ds(t, _L)][0]
        w = (r >> 7) << 7
        return jnp.where(r >= tail0, 0, w)

    def issue(w, slot):
        pltpu.async_copy(
            tab_hbm.at[:, pl.ds(pl.multiple_of(w, _W), _W)],
            bufs[slot], sems[slot])

    for q in range(_RING):
        issue(win_of(starts_v[pl.ds(q, _L)][0]), q)

    def runs(kk, carry):
        for q in range(_RING):
            kidx = kk * _RING + q
            sten = starts_v[pl.ds(kidx, _L)]
            st, en = sten[0], sten[1]
            pltpu.make_async_copy(
                tab_hbm.at[:, pl.ds(0, _W)], bufs[q], sems[q]).wait()

            def row(t, c):
                r = sv_v[pl.ds(t, _L)][0]
                j = jnp.where(r >= tail0, r - tail0, r & (_W - 1))
                col = jnp.full((_L,), j, jnp.int32)
                g0 = plsc.load_gather(bufs[q], [iota, col])
                g1 = plsc.load_gather(bufs[q], [iota + _L, col])
                t0 = plsc.load_gather(tail_v, [iota, col])
                t1 = plsc.load_gather(tail_v, [iota + _L, col])
                tl = r >= tail0
                e0 = jnp.where(tl, t0, g0)
                e1 = jnp.where(tl, t1, g1)
                rowi = jnp.full((_L,), t, jnp.int32)
                plsc.store_scatter(myrows_v, [rowi, iota], e0)
                plsc.store_scatter(myrows_v, [rowi, iota + _L], e1)
                return c
            lax.fori_loop(st, en, row, 0)

            @pl.when(kidx + _RING < kp)
            def _():
                issue(win_of(starts_v[pl.ds(kidx + _RING, _L)][0]), q)
        return carry
    lax.fori_loop(0, kp // _RING, runs, 0)

    pltpu.async_copy(myrows_v, stage_hbm.at[pl_v], sem_s).wait()


def _mf_kernel(su_hbm, pu_hbm, si_hbm, pi_hbm, ut_hbm, vt_hbm,
               utail_hbm, vtail_hbm,
               uet_hbm, iet_hbm, r_hbm,
               sv_v, sv2_v, pl_v, starts_v, myrows_v, slabu_v, slabv_v,
               utail_v, vtail_v, ucol_v, vcol_v, r_v,
               hu_hbm, hv_hbm, bufs_and_sems,
               *, bpw, dim, half, nu, nv):
    core = lax.axis_index("c")
    sub = lax.axis_index("s")
    base = core * half + sub * bpw
    bufs = bufs_and_sems[:_RING]
    sems = bufs_and_sems[_RING:2 * _RING]
    sem_s = bufs_and_sems[2 * _RING]

    pltpu.sync_copy(utail_hbm, utail_v)
    pltpu.sync_copy(vtail_hbm, vtail_v)

    pltpu.sync_copy(su_hbm.at[pl.ds(base, bpw)], sv_v.at[pl.ds(0, bpw)])
    sv_v[pl.ds(bpw, _L)] = jnp.zeros((_L,), jnp.int32)
    pltpu.sync_copy(pu_hbm.at[pl.ds(base, bpw)], pl_v)
    _run_table(ut_hbm, nu, sv_v, sv2_v, pl_v, starts_v, myrows_v,
               utail_v, bufs, sems, sem_s, hu_hbm, bpw=bpw)

    pltpu.sync_copy(si_hbm.at[pl.ds(base, bpw)], sv_v.at[pl.ds(0, bpw)])
    sv_v[pl.ds(bpw, _L)] = jnp.zeros((_L,), jnp.int32)
    pltpu.sync_copy(pi_hbm.at[pl.ds(base, bpw)], pl_v)
    _run_table(vt_hbm, nv, sv_v, sv2_v, pl_v, starts_v, myrows_v,
               vtail_v, bufs, sems, sem_s, hv_hbm, bpw=bpw)

    plsc.subcore_barrier()

    iota = lax.iota(jnp.int32, _L)
    for s in range(bpw // _SUB):
        off = base + s * _SUB
        pltpu.sync_copy(hu_hbm.at[pl.ds(off, _SUB)], slabu_v)
        pltpu.sync_copy(hv_hbm.at[pl.ds(off, _SUB)], slabv_v)
        for c in range(dim):
            colc = jnp.full((_L,), c, jnp.int32)

            def grp(g, carry):
                rows = g * _L + iota
                gu = plsc.load_gather(slabu_v, [rows, colc])
                gv = plsc.load_gather(slabv_v, [rows, colc])
                ucol_v[pl.ds(g * _L, _L)] = gu
                vcol_v[pl.ds(g * _L, _L)] = gv
                ro = s * _SUB + g * _L
                if c == 0:
                    r_v[pl.ds(ro, _L)] = gu * gv
                else:
                    r_v[pl.ds(ro, _L)] = r_v[pl.ds(ro, _L)] + gu * gv
                return carry
            lax.fori_loop(0, _SUB // _L, grp, 0)
            pltpu.sync_copy(ucol_v, uet_hbm.at[c, pl.ds(off, _SUB)])
            pltpu.sync_copy(vcol_v, iet_hbm.at[c, pl.ds(off, _SUB)])

    pltpu.sync_copy(r_v, r_hbm.at[pl.ds(base, bpw)])


def kernel(users, items, U, V):
    batch = users.shape[0]
    dim = U.shape[1]
    nrow_u = U.shape[0]
    nrow_v = V.shape[0]
    users = users.astype(jnp.int32)
    items = items.astype(jnp.int32)

    ut = U.T
    vt = V.T
    nu = (nrow_u // _W) * _W
    nv = (nrow_v // _W) * _W
    utail = jnp.zeros((dim, _W), jnp.float32).at[:, :nrow_u - nu].set(
        U[nu:].T)
    vtail = jnp.zeros((dim, _W), jnp.float32).at[:, :nrow_v - nv].set(
        V[nv:].T)

    half = batch // 2
    u2 = users.reshape(2, half)
    i2 = items.reshape(2, half)
    pu = jnp.argsort(u2, axis=1).astype(jnp.int32)
    pi = jnp.argsort(i2, axis=1).astype(jnp.int32)
    su = jnp.take_along_axis(u2, pu, axis=1).reshape(batch)
    si = jnp.take_along_axis(i2, pi, axis=1).reshape(batch)
    off = (jnp.arange(2, dtype=jnp.int32) * half)[:, None]
    pu = (pu + off).reshape(batch)
    pi = (pi + off).reshape(batch)

    info = plsc.get_sparse_core_info()
    num_workers = info.num_cores * info.num_subcores
    bpw = batch // num_workers

    mesh = plsc.VectorSubcoreMesh(core_axis_name="c", subcore_axis_name="s")

    scratch = [
        pltpu.VMEM((bpw + _L,), jnp.int32),
        pltpu.VMEM((bpw + _L,), jnp.int32),
        pltpu.VMEM((bpw,), jnp.int32),
        pltpu.VMEM((bpw + 2 * _L,), jnp.int32),
        pltpu.VMEM((bpw, _W), jnp.float32),
        pltpu.VMEM((_SUB, _W), jnp.float32),
        pltpu.VMEM((_SUB, _W), jnp.float32),
        pltpu.VMEM((dim, _W), jnp.float32),
        pltpu.VMEM((dim, _W), jnp.float32),
        pltpu.VMEM((_SUB,), jnp.float32),
        pltpu.VMEM((_SUB,), jnp.float32),
        pltpu.VMEM((bpw,), jnp.float32),
        pltpu.HBM((batch, _W), jnp.float32),
        pltpu.HBM((batch, _W), jnp.float32),
    ]
    scratch += [pltpu.VMEM((dim, _W), jnp.float32) for _ in range(_RING)]
    scratch += [pltpu.SemaphoreType.DMA for _ in range(_RING + 1)]

    def body(su_h, pu_h, si_h, pi_h, ut_h, vt_h, utl_h, vtl_h,
             uet_h, iet_h, r_h,
             sv, sv2, plv, starts, myrows, slabu, slabv, utl, vtl,
             uc, vc, rr, hu, hv, *ring):
        _mf_kernel(su_h, pu_h, si_h, pi_h, ut_h, vt_h, utl_h, vtl_h,
                   uet_h, iet_h, r_h,
                   sv, sv2, plv, starts, myrows, slabu, slabv, utl, vtl,
                   uc, vc, rr, hu, hv, list(ring),
                   bpw=bpw, dim=dim, half=half, nu=nu, nv=nv)

    mf = pl.kernel(
        body,
        out_type=(
            jax.ShapeDtypeStruct((dim, batch), jnp.float32),
            jax.ShapeDtypeStruct((dim, batch), jnp.float32),
            jax.ShapeDtypeStruct((batch,), jnp.float32),
        ),
        mesh=mesh,
        compiler_params=pltpu.CompilerParams(needs_layout_passes=False,
                                             use_tc_tiling_on_sc=True),
        scratch_types=scratch,
    )

    uet, iet, r_hats = mf(su, pu, si, pi, ut, vt, utail, vtail)
    return (uet.T, iet.T, r_hats)


# speedup vs baseline: 1.2937x; 1.2937x over previous
import functools

import jax
import jax.numpy as jnp
from jax import lax
from jax.experimental import pallas as pl
from jax.experimental.pallas import tpu as pltpu
from jax.experimental.pallas import tpu_sc as plsc

_L = 16
_RING = 8
_W = 128


def _mf_kernel(users_hbm, items_hbm, ut_hbm, vt_hbm, utail_hbm, vtail_hbm,
               uet_hbm, iet_hbm, r_hbm,
               uidx_v, iidx_v, warr_v, utail_v, vtail_v,
               outu_v, outv_v, r_v, bufs_and_sems,
               *, bpw, dim, num_cores, nu, nv):
    wid = lax.axis_index("s") * num_cores + lax.axis_index("c")
    base = wid * bpw
    bufs = bufs_and_sems[:_RING]
    sems = bufs_and_sems[_RING:]

    iota = lax.iota(jnp.int32, _L)

    pltpu.sync_copy(users_hbm.at[pl.ds(base, bpw)], uidx_v)
    pltpu.sync_copy(items_hbm.at[pl.ds(base, bpw)], iidx_v)
    pltpu.sync_copy(utail_hbm, utail_v)
    pltpu.sync_copy(vtail_hbm, vtail_v)

    def run_table(tab_hbm, idx_v, tail_v, out_v, tail0, is_v):
        nchunk = bpw // _L

        def wprep(k, carry):
            v = idx_v[pl.ds(k * _L, _L)]
            w = (v >> 7) << 7
            w = jnp.where(v >= tail0, 0, w)
            warr_v[pl.ds(k * _L, _L)] = w
            return carry
        lax.fori_loop(0, nchunk, wprep, 0)
        warr_v[pl.ds(bpw, _L)] = jnp.zeros((_L,), jnp.int32)

        def issue(row_w, slot):
            pltpu.async_copy(
                tab_hbm.at[:, pl.ds(pl.multiple_of(row_w, _W), _W)],
                bufs[slot], sems[slot])

        w0 = warr_v[pl.ds(0, _L)]
        for l in range(_RING):
            issue(w0[l], l)

        def chunk(jj, carry):
            rvec = idx_v[pl.ds(jj * _L, _L)]
            wnext = warr_v[pl.ds(jj * _L + _RING, _L)]
            racc = r_v[pl.ds(jj * _L, _L)]
            for l in range(_L):
                i = jj * _L + l
                slot = l % _RING
                r = rvec[l]
                pltpu.make_async_copy(
                    tab_hbm.at[:, pl.ds(0, _W)], bufs[slot],
                    sems[slot]).wait()
                j = jnp.where(r >= tail0, r - tail0, r & (_W - 1))
                col = jnp.full((_L,), j, jnp.int32)
                g0 = plsc.load_gather(bufs[slot], [iota, col])
                g1 = plsc.load_gather(bufs[slot], [iota + _L, col])
                t0 = plsc.load_gather(tail_v, [iota, col])
                t1 = plsc.load_gather(tail_v, [iota + _L, col])
                tl = r >= tail0
                e0 = jnp.where(tl, t0, g0)
                e1 = jnp.where(tl, t1, g1)
                coli = jnp.full((_L,), i, jnp.int32)
                plsc.store_scatter(out_v, [iota, coli], e0)
                plsc.store_scatter(out_v, [iota + _L, coli], e1)
                if is_v:
                    u0 = plsc.load_gather(outu_v, [iota, coli])
                    u1 = plsc.load_gather(outu_v, [iota + _L, coli])
                    s = jnp.sum(u0 * e0 + u1 * e1)
                    racc = jnp.where(iota == l, s, racc)
                @pl.when(i + _RING < bpw)
                def _():
                    issue(wnext[l], slot)
            if is_v:
                r_v[pl.ds(jj * _L, _L)] = racc
            return carry
        lax.fori_loop(0, nchunk, chunk, 0)

    run_table(ut_hbm, uidx_v, utail_v, outu_v, nu, False)
    run_table(vt_hbm, iidx_v, vtail_v, outv_v, nv, True)

    pltpu.sync_copy(outu_v, uet_hbm.at[:, pl.ds(base, bpw)])
    pltpu.sync_copy(outv_v, iet_hbm.at[:, pl.ds(base, bpw)])
    pltpu.sync_copy(r_v, r_hbm.at[pl.ds(base, bpw)])


def kernel(users, items, U, V):
    batch = users.shape[0]
    dim = U.shape[1]
    nrow_u = U.shape[0]
    nrow_v = V.shape[0]
    users = users.astype(jnp.int32)
    items = items.astype(jnp.int32)

    ut = U.T
    vt = V.T
    nu = (nrow_u // _W) * _W
    nv = (nrow_v // _W) * _W
    utail = jnp.zeros((dim, _W), jnp.float32).at[:, :nrow_u - nu].set(
        U[nu:].T)
    vtail = jnp.zeros((dim, _W), jnp.float32).at[:, :nrow_v - nv].set(
        V[nv:].T)

    info = plsc.get_sparse_core_info()
    num_workers = info.num_cores * info.num_subcores
    bpw = batch // num_workers

    mesh = plsc.VectorSubcoreMesh(core_axis_name="c", subcore_axis_name="s")

    scratch = [
        pltpu.VMEM((bpw,), jnp.int32),
        pltpu.VMEM((bpw,), jnp.int32),
        pltpu.VMEM((bpw + 2 * _RING,), jnp.int32),
        pltpu.VMEM((dim, _W), jnp.float32),
        pltpu.VMEM((dim, _W), jnp.float32),
        pltpu.VMEM((dim, bpw), jnp.float32),
        pltpu.VMEM((dim, bpw), jnp.float32),
        pltpu.VMEM((bpw,), jnp.float32),
    ]
    scratch += [pltpu.VMEM((dim, _W), jnp.float32) for _ in range(_RING)]
    scratch += [pltpu.SemaphoreType.DMA for _ in range(_RING)]

    def body(users_h, items_h, ut_h, vt_h, utail_h, vtail_h,
             uet_h, iet_h, r_h, uidx, iidx, warr, utl, vtl,
             outu, outv, rr, *ring):
        _mf_kernel(users_h, items_h, ut_h, vt_h, utail_h, vtail_h,
                   uet_h, iet_h, r_h, uidx, iidx, warr, utl, vtl,
                   outu, outv, rr, list(ring),
                   bpw=bpw, dim=dim, num_cores=info.num_cores,
                   nu=nu, nv=nv)

    mf = pl.kernel(
        body,
        out_type=(
            jax.ShapeDtypeStruct((dim, batch), jnp.float32),
            jax.ShapeDtypeStruct((dim, batch), jnp.float32),
            jax.ShapeDtypeStruct((batch,), jnp.float32),
        ),
        mesh=mesh,
        compiler_params=pltpu.CompilerParams(needs_layout_passes=False,
                                             use_tc_tiling_on_sc=True),
        scratch_types=scratch,
    )

    uet, iet, r_hats = mf(users, items, ut, vt, utail, vtail)
    return (uet.T, iet.T, r_hats)
